# baseline (speedup 1.0000x reference)
import jax
import jax.numpy as jnp
from jax import lax
from jax.experimental import pallas as pl
from jax.experimental.pallas import tpu as pltpu

N_ROWS = 16384
N_COLS = 1024
CH = 512
N_CHUNKS = N_ROWS // CH
NBUF = 8


def _fma_kernel(x_hbm, mu_ref, sigma_ref, o_hbm, *scr):
    ins = scr[0:NBUF]
    ots = scr[NBUF:2 * NBUF]
    isems = scr[2 * NBUF:3 * NBUF]
    osems = scr[3 * NBUF:4 * NBUF]

    for b in range(NBUF):
        pltpu.async_copy(x_hbm.at[pl.ds(b * CH, CH)], ins[b], isems[b])

    def step(i, carry):
        g = i * NBUF
        for b in range(NBUF):
            k = g + b
            row0 = k * CH
            pltpu.make_async_copy(x_hbm.at[pl.ds(row0, CH)],
                                  ins[b], isems[b]).wait()

            @pl.when(i >= 1)
            def _():
                pltpu.make_async_copy(
                    ots[b], o_hbm.at[pl.ds(row0 - NBUF * CH, CH)],
                    osems[b]).wait()

            ots[b][...] = mu_ref[...] + ins[b][...] * sigma_ref[...]
            pltpu.async_copy(ots[b], o_hbm.at[pl.ds(row0, CH)], osems[b])

            @pl.when(i <= N_CHUNKS // NBUF - 2)
            def _():
                pltpu.async_copy(x_hbm.at[pl.ds(row0 + NBUF * CH, CH)],
                                 ins[b], isems[b])
        return carry

    lax.fori_loop(0, N_CHUNKS // NBUF, step, 0)

    for b in range(NBUF):
        row0 = (N_CHUNKS - NBUF + b) * CH
        pltpu.make_async_copy(ots[b], o_hbm.at[pl.ds(row0, CH)],
                              osems[b]).wait()


def kernel(x, mu, sigma):
    mu2 = mu.reshape(1, N_COLS)
    sigma2 = sigma.reshape(1, N_COLS)
    scratch = (
        [pltpu.VMEM((CH, N_COLS), jnp.float32)] * (2 * NBUF)
        + [pltpu.SemaphoreType.DMA] * (2 * NBUF)
    )
    return pl.pallas_call(
        _fma_kernel,
        in_specs=[
            pl.BlockSpec(memory_space=pl.ANY),
            pl.BlockSpec(memory_space=pltpu.VMEM),
            pl.BlockSpec(memory_space=pltpu.VMEM),
        ],
        out_specs=pl.BlockSpec(memory_space=pl.ANY),
        out_shape=jax.ShapeDtypeStruct((N_ROWS, N_COLS), x.dtype),
        scratch_shapes=scratch,
    )(x, mu2, sigma2)


# speedup vs baseline: 1.0172x; 1.0172x over previous
import jax
import jax.numpy as jnp
from jax.experimental import pallas as pl

N_ROWS = 16384
N_COLS = 1024
BM = 2048


def _fma_kernel(x_ref, mu_ref, sigma_ref, o_ref):
    o_ref[...] = mu_ref[...] + x_ref[...] * sigma_ref[...]


def kernel(x, mu, sigma):
    mu2 = mu.reshape(1, N_COLS)
    sigma2 = sigma.reshape(1, N_COLS)
    return pl.pallas_call(
        _fma_kernel,
        grid=(N_ROWS // BM,),
        in_specs=[
            pl.BlockSpec((BM, N_COLS), lambda i: (i, 0)),
            pl.BlockSpec((1, N_COLS), lambda i: (0, 0)),
            pl.BlockSpec((1, N_COLS), lambda i: (0, 0)),
        ],
        out_specs=pl.BlockSpec((BM, N_COLS), lambda i: (i, 0)),
        out_shape=jax.ShapeDtypeStruct((N_ROWS, N_COLS), x.dtype),
    )(x, mu2, sigma2)


# speedup vs baseline: 1.0521x; 1.0344x over previous
import jax
import jax.numpy as jnp
from jax.experimental import pallas as pl

N_ROWS = 16384
N_COLS = 1024
BM = 3072


def _fma_kernel(x_ref, mu_ref, sigma_ref, o_ref):
    o_ref[...] = mu_ref[...] + x_ref[...] * sigma_ref[...]


def kernel(x, mu, sigma):
    mu2 = mu.reshape(1, N_COLS)
    sigma2 = sigma.reshape(1, N_COLS)
    return pl.pallas_call(
        _fma_kernel,
        grid=(pl.cdiv(N_ROWS, BM),),
        in_specs=[
            pl.BlockSpec((BM, N_COLS), lambda i: (i, 0)),
            pl.BlockSpec((1, N_COLS), lambda i: (0, 0)),
            pl.BlockSpec((1, N_COLS), lambda i: (0, 0)),
        ],
        out_specs=pl.BlockSpec((BM, N_COLS), lambda i: (i, 0)),
        out_shape=jax.ShapeDtypeStruct((N_ROWS, N_COLS), x.dtype),
    )(x, mu2, sigma2)


# speedup vs baseline: 1.0541x; 1.0018x over previous
import jax
import jax.numpy as jnp
from jax.experimental import pallas as pl

N_ROWS = 16384
N_COLS = 1024
BM = 3584


def _fma_kernel(x_ref, mu_ref, sigma_ref, o_ref):
    o_ref[...] = mu_ref[...] + x_ref[...] * sigma_ref[...]


def kernel(x, mu, sigma):
    mu2 = mu.reshape(1, N_COLS)
    sigma2 = sigma.reshape(1, N_COLS)
    return pl.pallas_call(
        _fma_kernel,
        grid=(pl.cdiv(N_ROWS, BM),),
        in_specs=[
            pl.BlockSpec((BM, N_COLS), lambda i: (i, 0)),
            pl.BlockSpec((1, N_COLS), lambda i: (0, 0)),
            pl.BlockSpec((1, N_COLS), lambda i: (0, 0)),
        ],
        out_specs=pl.BlockSpec((BM, N_COLS), lambda i: (i, 0)),
        out_shape=jax.ShapeDtypeStruct((N_ROWS, N_COLS), x.dtype),
    )(x, mu2, sigma2)
